# baseline (speedup 1.0000x reference)
import functools

import jax
import jax.numpy as jnp
from jax import lax
from jax.experimental import pallas as pl
from jax.experimental.pallas import tpu as pltpu
from jax.experimental.pallas import tpu_sc as plsc

N_CLS = 128
N_CTX = 32
HALF = N_CTX // 2
N_EXPERTS = 64
TOP_K = 4
CTX_DIM = 768
SEQ_LEN = 77
SUF_LEN = SEQ_LEN - 1 - N_CTX
EXP_FLAT = (HALF - 1) * CTX_DIM
N_SEG = HALF - 1
SEG = CTX_DIM
LANES = 16

NC = 2
NS = 16


def _gate_body(rad_ref, w_gate_ref, shared_ref, ws_w_ref, ws_b_ref,
               idx_ref, gates_ref, ctx_s_ref, aux_ref):
    ctx_s = lax.dot_general(shared_ref[...], ws_w_ref[...],
                            (((1,), (1,)), ((), ())),
                            preferred_element_type=jnp.float32)
    ctx_s_ref[...] = ctx_s + ws_b_ref[...]

    logits = lax.dot_general(rad_ref[...], w_gate_ref[...],
                             (((1,), (0,)), ((), ())),
                             preferred_element_type=jnp.float32)
    iota = lax.broadcasted_iota(jnp.int32, (1, N_EXPERTS), 1)
    v = logits
    vals, idxs = [], []
    for _ in range(TOP_K):
        s = jnp.max(v)
        e = jnp.min(jnp.where(v == s, iota, N_EXPERTS))
        vals.append(s)
        idxs.append(e)
        v = jnp.where(iota == e, -jnp.inf, v)

    m = vals[0]
    exps = [jnp.exp(val - m) for val in vals]
    tot = exps[0] + exps[1] + exps[2] + exps[3]
    gs = [ex / tot for ex in exps]

    g64 = jnp.zeros((1, N_EXPERTS), jnp.float32)
    for k in range(TOP_K):
        g64 = jnp.where(iota == idxs[k], gs[k], g64)
    s1 = jnp.sum(g64)
    s2 = jnp.sum(g64 * g64)
    mean = s1 / N_EXPERTS
    var = (s2 - N_EXPERTS * mean * mean) / (N_EXPERTS - 1)
    aux_ref[...] = jnp.full((1, 1), var / (mean * mean + 1e-10), jnp.float32)

    iota16 = lax.broadcasted_iota(jnp.int32, (1, LANES), 1)
    iv = jnp.zeros((1, LANES), jnp.int32)
    for k in range(TOP_K):
        iv = jnp.where(iota16 == k, idxs[k], iv)
    idx_ref[...] = iv
    row4 = lax.broadcasted_iota(jnp.int32, (TOP_K, LANES), 0)
    gsp = jnp.zeros((TOP_K, LANES), jnp.float32)
    for k in range(TOP_K):
        gsp = jnp.where(row4 == k, gs[k], gsp)
    gates_ref[...] = gsp


def _mix_body(ctx_hbm, idx_hbm, gates_hbm, out_hbm,
              idx_v, g_v, ids_v, rows_v, acc_v, sem):
    cid = lax.axis_index("c")
    sid = lax.axis_index("s")
    wid = cid * NS + sid

    @pl.when(wid < N_SEG)
    def _():
        pltpu.sync_copy(idx_hbm, idx_v)
        pltpu.sync_copy(gates_hbm, g_v)
        iv = idx_v[...]
        ids_v[...] = iv * N_SEG + jnp.broadcast_to(wid, (LANES,))
        pltpu.async_copy(ctx_hbm.at[ids_v.at[pl.ds(0, TOP_K)]],
                         rows_v, sem).wait()
        gks = [g_v[k] for k in range(TOP_K)]
        for j in range(SEG // LANES):
            sl = pl.ds(j * LANES, LANES)
            acc = gks[0] * rows_v[0, sl]
            for k in range(1, TOP_K):
                acc = acc + gks[k] * rows_v[k, sl]
            acc_v[sl] = acc
        pltpu.sync_copy(acc_v, out_hbm.at[pl.ds(wid * SEG, SEG)])


def _asm_body(prefix_ref, suffix_ref, ctxg_ref, mix_ref, ctxs_ref, out_ref):
    blk = out_ref.shape[0]
    out_ref[:, 0:1, :] = prefix_ref[...]
    out_ref[:, 1:1 + HALF, :] = jnp.broadcast_to(
        ctxg_ref[...][None], (blk, HALF, CTX_DIM))
    out_ref[:, 1 + HALF:N_CTX, :] = jnp.broadcast_to(
        mix_ref[...][None], (blk, HALF - 1, CTX_DIM))
    out_ref[:, N_CTX:N_CTX + 1, :] = jnp.broadcast_to(
        ctxs_ref[...][None], (blk, 1, CTX_DIM))
    out_ref[:, N_CTX + 1:, :] = suffix_ref[...]


def _gating(rad, shared, ctx_g, w_gate, ws_w, ws_b):
    del ctx_g
    return pl.pallas_call(
        _gate_body,
        out_shape=(
            jax.ShapeDtypeStruct((1, LANES), jnp.int32),
            jax.ShapeDtypeStruct((TOP_K, LANES), jnp.float32),
            jax.ShapeDtypeStruct((1, CTX_DIM), jnp.float32),
            jax.ShapeDtypeStruct((1, 1), jnp.float32),
        ),
    )(rad, w_gate, shared, ws_w, ws_b)


def _mix_call():
    return pl.kernel(
        _mix_body,
        out_type=jax.ShapeDtypeStruct((EXP_FLAT,), jnp.float32),
        mesh=plsc.VectorSubcoreMesh(core_axis_name="c", subcore_axis_name="s"),
        scratch_types=[
            pltpu.VMEM((LANES,), jnp.int32),
            pltpu.VMEM((TOP_K, LANES), jnp.float32),
            pltpu.VMEM((LANES,), jnp.int32),
            pltpu.VMEM((TOP_K, SEG), jnp.float32),
            pltpu.VMEM((SEG,), jnp.float32),
            pltpu.SemaphoreType.DMA,
        ],
    )


BLK = 16


def _assemble(token_prefix, token_suffix, ctx_g, mix, ctx_s):
    return pl.pallas_call(
        _asm_body,
        grid=(N_CLS // BLK,),
        in_specs=[
            pl.BlockSpec((BLK, 1, CTX_DIM), lambda i: (i, 0, 0)),
            pl.BlockSpec((BLK, SUF_LEN, CTX_DIM), lambda i: (i, 0, 0)),
            pl.BlockSpec((HALF, CTX_DIM), lambda i: (0, 0)),
            pl.BlockSpec((HALF - 1, CTX_DIM), lambda i: (0, 0)),
            pl.BlockSpec((1, CTX_DIM), lambda i: (0, 0)),
        ],
        out_specs=pl.BlockSpec((BLK, SEQ_LEN, CTX_DIM), lambda i: (i, 0, 0)),
        out_shape=jax.ShapeDtypeStruct((N_CLS, SEQ_LEN, CTX_DIM), jnp.float32),
    )(token_prefix, token_suffix, ctx_g, mix, ctx_s)


def kernel(rad, shared, ctx_g, ctx_c, Ws_w, Ws_b, w_gate,
           token_prefix, token_suffix, tokenized_prompts):
    idx16, gates16, ctx_s, aux = _gating(
        rad, shared, ctx_g, w_gate, Ws_w, Ws_b.reshape(1, CTX_DIM))
    mix_flat = _mix_call()(ctx_c, idx16.reshape(LANES), gates16)
    mix = mix_flat.reshape(HALF - 1, CTX_DIM)
    prompts = _assemble(token_prefix, token_suffix, ctx_g, mix, ctx_s)
    return prompts, tokenized_prompts, aux.reshape(())


# speedup vs baseline: 1.0006x; 1.0006x over previous
import jax
import jax.numpy as jnp
from jax import lax
from jax.experimental import pallas as pl
from jax.experimental.pallas import tpu as pltpu
from jax.experimental.pallas import tpu_sc as plsc

N_CLS = 128
N_CTX = 32
HALF = N_CTX // 2
N_EXPERTS = 64
TOP_K = 4
CTX_DIM = 768
SEQ_LEN = 77
SUF_LEN = SEQ_LEN - 1 - N_CTX
EXP_FLAT = (HALF - 1) * CTX_DIM
N_SEG = HALF - 1
SEG = CTX_DIM
LANES = 16

NC = 2
NS = 16


def _gate_body(rad_ref, w_gate_ref, shared_ref, ws_w_ref, ws_b_ref,
               idx_ref, gates_ref, ctx_s_ref, aux_ref):
    ctx_s = lax.dot_general(shared_ref[...], ws_w_ref[...],
                            (((1,), (1,)), ((), ())),
                            preferred_element_type=jnp.float32)
    ctx_s_ref[...] = ctx_s + ws_b_ref[...]

    logits = lax.dot_general(rad_ref[...], w_gate_ref[...],
                             (((1,), (0,)), ((), ())),
                             preferred_element_type=jnp.float32)
    iota = lax.broadcasted_iota(jnp.int32, (1, N_EXPERTS), 1)
    v = logits
    vals, idxs = [], []
    for _ in range(TOP_K):
        s = jnp.max(v)
        e = jnp.min(jnp.where(v == s, iota, N_EXPERTS))
        vals.append(s)
        idxs.append(e)
        v = jnp.where(iota == e, -jnp.inf, v)

    m = vals[0]
    exps = [jnp.exp(val - m) for val in vals]
    tot = exps[0] + exps[1] + exps[2] + exps[3]
    gs = [ex / tot for ex in exps]

    g64 = jnp.zeros((1, N_EXPERTS), jnp.float32)
    for k in range(TOP_K):
        g64 = jnp.where(iota == idxs[k], gs[k], g64)
    s1 = jnp.sum(g64)
    s2 = jnp.sum(g64 * g64)
    mean = s1 / N_EXPERTS
    var = (s2 - N_EXPERTS * mean * mean) / (N_EXPERTS - 1)
    aux_ref[...] = jnp.full((1, 1), var / (mean * mean + 1e-10), jnp.float32)

    iota16 = lax.broadcasted_iota(jnp.int32, (1, LANES), 1)
    iv = jnp.zeros((1, LANES), jnp.int32)
    for k in range(TOP_K):
        iv = jnp.where(iota16 == k, idxs[k], iv)
    idx_ref[...] = iv
    row4 = lax.broadcasted_iota(jnp.int32, (TOP_K, LANES), 0)
    gsp = jnp.zeros((TOP_K, LANES), jnp.float32)
    for k in range(TOP_K):
        gsp = jnp.where(row4 == k, gs[k], gsp)
    gates_ref[...] = gsp


def _mix_body(ctx_hbm, idx_hbm, gates_hbm, out_hbm,
              idx_v, g_v, ids_v, rows_v, acc_v, sem):
    cid = lax.axis_index("c")
    sid = lax.axis_index("s")
    wid = cid * NS + sid

    @pl.when(wid < N_SEG)
    def _():
        pltpu.sync_copy(idx_hbm, idx_v)
        pltpu.sync_copy(gates_hbm, g_v)
        iv = idx_v[...]
        ids_v[...] = iv * N_SEG + jnp.broadcast_to(wid, (LANES,))
        pltpu.async_copy(ctx_hbm.at[ids_v.at[pl.ds(0, TOP_K)]],
                         rows_v, sem).wait()
        gks = [g_v[k] for k in range(TOP_K)]
        for j in range(SEG // LANES):
            sl = pl.ds(j * LANES, LANES)
            acc = gks[0] * rows_v[0, sl]
            for k in range(1, TOP_K):
                acc = acc + gks[k] * rows_v[k, sl]
            acc_v[sl] = acc
        pltpu.sync_copy(acc_v, out_hbm.at[pl.ds(wid * SEG, SEG)])


def _asm_body(prefix_ref, suffix_ref, ctxg_ref, mix_ref, ctxs_ref, out_ref):
    blk = out_ref.shape[0]
    out_ref[:, 0:1, :] = prefix_ref[...]
    out_ref[:, 1:1 + HALF, :] = jnp.broadcast_to(
        ctxg_ref[...][None], (blk, HALF, CTX_DIM))
    out_ref[:, 1 + HALF:N_CTX, :] = jnp.broadcast_to(
        mix_ref[...][None], (blk, HALF - 1, CTX_DIM))
    out_ref[:, N_CTX:N_CTX + 1, :] = jnp.broadcast_to(
        ctxs_ref[...][None], (blk, 1, CTX_DIM))
    out_ref[:, N_CTX + 1:, :] = suffix_ref[...]


def _gating(rad, shared, ctx_g, w_gate, ws_w, ws_b):
    del ctx_g
    return pl.pallas_call(
        _gate_body,
        out_shape=(
            jax.ShapeDtypeStruct((1, LANES), jnp.int32),
            jax.ShapeDtypeStruct((TOP_K, LANES), jnp.float32),
            jax.ShapeDtypeStruct((1, CTX_DIM), jnp.float32),
            jax.ShapeDtypeStruct((1, 1), jnp.float32),
        ),
    )(rad, w_gate, shared, ws_w, ws_b)


def _mix_call():
    return pl.kernel(
        _mix_body,
        out_type=jax.ShapeDtypeStruct((EXP_FLAT,), jnp.float32),
        mesh=plsc.VectorSubcoreMesh(core_axis_name="c", subcore_axis_name="s"),
        scratch_types=[
            pltpu.VMEM((LANES,), jnp.int32),
            pltpu.VMEM((TOP_K, LANES), jnp.float32),
            pltpu.VMEM((LANES,), jnp.int32),
            pltpu.VMEM((TOP_K, SEG), jnp.float32),
            pltpu.VMEM((SEG,), jnp.float32),
            pltpu.SemaphoreType.DMA,
        ],
    )


BLK = 16


def _assemble(token_prefix, token_suffix, ctx_g, mix, ctx_s):
    return pl.pallas_call(
        _asm_body,
        grid=(N_CLS // BLK,),
        in_specs=[
            pl.BlockSpec((BLK, 1, CTX_DIM), lambda i: (i, 0, 0)),
            pl.BlockSpec((BLK, SUF_LEN, CTX_DIM), lambda i: (i, 0, 0)),
            pl.BlockSpec((HALF, CTX_DIM), lambda i: (0, 0)),
            pl.BlockSpec((HALF - 1, CTX_DIM), lambda i: (0, 0)),
            pl.BlockSpec((1, CTX_DIM), lambda i: (0, 0)),
        ],
        out_specs=pl.BlockSpec((BLK, SEQ_LEN, CTX_DIM), lambda i: (i, 0, 0)),
        out_shape=jax.ShapeDtypeStruct((N_CLS, SEQ_LEN, CTX_DIM), jnp.float32),
    )(token_prefix, token_suffix, ctx_g, mix, ctx_s)


def kernel(rad, shared, ctx_g, ctx_c, Ws_w, Ws_b, w_gate,
           token_prefix, token_suffix, tokenized_prompts):
    idx16, gates16, ctx_s, aux = _gating(
        rad, shared, ctx_g, w_gate, Ws_w, Ws_b.reshape(1, CTX_DIM))
    mix_flat = _mix_call()(ctx_c, idx16.reshape(LANES), gates16)
    mix = mix_flat.reshape(HALF - 1, CTX_DIM)
    prompts = _assemble(token_prefix, token_suffix, ctx_g, mix, ctx_s)
    return prompts, tokenized_prompts, aux.reshape(())


# speedup vs baseline: 1.0219x; 1.0213x over previous
import jax
import jax.numpy as jnp
from jax import lax
from jax.experimental import pallas as pl
from jax.experimental.pallas import tpu as pltpu
from jax.experimental.pallas import tpu_sc as plsc

N_CLS = 128
N_CTX = 32
HALF = N_CTX // 2
N_EXPERTS = 64
TOP_K = 4
CTX_DIM = 768
SEQ_LEN = 77
SUF_LEN = SEQ_LEN - 1 - N_CTX
EXP_FLAT = (HALF - 1) * CTX_DIM
N_SEG = HALF - 1
SEG = CTX_DIM
LANES = 16

NC = 2
NS = 16


def _gate_body(rad_ref, w_gate_ref, shared_ref, ws_w_ref, ws_b_ref,
               idx_ref, gates_ref, ctx_s_ref, aux_ref):
    ctx_s = lax.dot_general(shared_ref[...], ws_w_ref[...],
                            (((1,), (1,)), ((), ())),
                            preferred_element_type=jnp.float32)
    ctx_s_ref[...] = ctx_s + ws_b_ref[...]

    logits = lax.dot_general(rad_ref[...], w_gate_ref[...],
                             (((1,), (0,)), ((), ())),
                             preferred_element_type=jnp.float32)
    iota = lax.broadcasted_iota(jnp.int32, (1, N_EXPERTS), 1)
    v = logits
    vals, idxs = [], []
    for _ in range(TOP_K):
        s = jnp.max(v)
        e = jnp.min(jnp.where(v == s, iota, N_EXPERTS))
        vals.append(s)
        idxs.append(e)
        v = jnp.where(iota == e, -jnp.inf, v)

    m = vals[0]
    exps = [jnp.exp(val - m) for val in vals]
    tot = exps[0] + exps[1] + exps[2] + exps[3]
    gs = [ex / tot for ex in exps]

    g64 = jnp.zeros((1, N_EXPERTS), jnp.float32)
    for k in range(TOP_K):
        g64 = jnp.where(iota == idxs[k], gs[k], g64)
    s1 = jnp.sum(g64)
    s2 = jnp.sum(g64 * g64)
    mean = s1 / N_EXPERTS
    var = (s2 - N_EXPERTS * mean * mean) / (N_EXPERTS - 1)
    aux_ref[...] = jnp.full((1, 1), var / (mean * mean + 1e-10), jnp.float32)

    iota16 = lax.broadcasted_iota(jnp.int32, (1, LANES), 1)
    iv = jnp.zeros((1, LANES), jnp.int32)
    for k in range(TOP_K):
        iv = jnp.where(iota16 == k, idxs[k], iv)
    idx_ref[...] = iv
    row4 = lax.broadcasted_iota(jnp.int32, (TOP_K, LANES), 0)
    gsp = jnp.zeros((TOP_K, LANES), jnp.float32)
    for k in range(TOP_K):
        gsp = jnp.where(row4 == k, gs[k], gsp)
    gates_ref[...] = gsp


def _mix_body(ctx_hbm, idx_hbm, gates_hbm, out_hbm,
              idx_v, g_v, ids_v, rows_v, acc_v, sem):
    cid = lax.axis_index("c")
    sid = lax.axis_index("s")
    wid = cid * NS + sid

    @pl.when(wid < N_SEG)
    def _():
        pltpu.sync_copy(idx_hbm, idx_v)
        pltpu.sync_copy(gates_hbm, g_v)
        iv = idx_v[...]
        ids_v[...] = iv * N_SEG + jnp.broadcast_to(wid, (LANES,))
        pltpu.async_copy(ctx_hbm.at[ids_v.at[pl.ds(0, TOP_K)]],
                         rows_v, sem).wait()
        gks = [g_v[k] for k in range(TOP_K)]
        for j in range(SEG // LANES):
            sl = pl.ds(j * LANES, LANES)
            acc = gks[0] * rows_v[0, sl]
            for k in range(1, TOP_K):
                acc = acc + gks[k] * rows_v[k, sl]
            acc_v[sl] = acc
        pltpu.sync_copy(acc_v, out_hbm.at[pl.ds(wid * SEG, SEG)])


def _asm_body(prefix_ref, suffix_ref, ctxg_ref, mix_ref, ctxs_ref, out_ref):
    blk = out_ref.shape[0]
    out_ref[:, 0:1, :] = prefix_ref[...]
    out_ref[:, 1:1 + HALF, :] = jnp.broadcast_to(
        ctxg_ref[...][None], (blk, HALF, CTX_DIM))
    out_ref[:, 1 + HALF:N_CTX, :] = jnp.broadcast_to(
        mix_ref[...][None], (blk, HALF - 1, CTX_DIM))
    out_ref[:, N_CTX:N_CTX + 1, :] = jnp.broadcast_to(
        ctxs_ref[...][None], (blk, 1, CTX_DIM))
    out_ref[:, N_CTX + 1:, :] = suffix_ref[...]


def _gating(rad, shared, ctx_g, w_gate, ws_w, ws_b):
    del ctx_g
    return pl.pallas_call(
        _gate_body,
        out_shape=(
            jax.ShapeDtypeStruct((1, LANES), jnp.int32),
            jax.ShapeDtypeStruct((TOP_K, LANES), jnp.float32),
            jax.ShapeDtypeStruct((1, CTX_DIM), jnp.float32),
            jax.ShapeDtypeStruct((1, 1), jnp.float32),
        ),
    )(rad, w_gate, shared, ws_w, ws_b)


def _mix_call():
    return pl.kernel(
        _mix_body,
        out_type=jax.ShapeDtypeStruct((EXP_FLAT,), jnp.float32),
        mesh=plsc.VectorSubcoreMesh(core_axis_name="c", subcore_axis_name="s"),
        scratch_types=[
            pltpu.VMEM((LANES,), jnp.int32),
            pltpu.VMEM((TOP_K, LANES), jnp.float32),
            pltpu.VMEM((LANES,), jnp.int32),
            pltpu.VMEM((TOP_K, SEG), jnp.float32),
            pltpu.VMEM((SEG,), jnp.float32),
            pltpu.SemaphoreType.DMA,
        ],
    )


BLK = 32


def _assemble(token_prefix, token_suffix, ctx_g, mix, ctx_s):
    return pl.pallas_call(
        _asm_body,
        grid=(N_CLS // BLK,),
        in_specs=[
            pl.BlockSpec((BLK, 1, CTX_DIM), lambda i: (i, 0, 0)),
            pl.BlockSpec((BLK, SUF_LEN, CTX_DIM), lambda i: (i, 0, 0)),
            pl.BlockSpec((HALF, CTX_DIM), lambda i: (0, 0)),
            pl.BlockSpec((HALF - 1, CTX_DIM), lambda i: (0, 0)),
            pl.BlockSpec((1, CTX_DIM), lambda i: (0, 0)),
        ],
        out_specs=pl.BlockSpec((BLK, SEQ_LEN, CTX_DIM), lambda i: (i, 0, 0)),
        out_shape=jax.ShapeDtypeStruct((N_CLS, SEQ_LEN, CTX_DIM), jnp.float32),
    )(token_prefix, token_suffix, ctx_g, mix, ctx_s)


def kernel(rad, shared, ctx_g, ctx_c, Ws_w, Ws_b, w_gate,
           token_prefix, token_suffix, tokenized_prompts):
    idx16, gates16, ctx_s, aux = _gating(
        rad, shared, ctx_g, w_gate, Ws_w, Ws_b.reshape(1, CTX_DIM))
    mix_flat = _mix_call()(ctx_c, idx16.reshape(LANES), gates16)
    mix = mix_flat.reshape(HALF - 1, CTX_DIM)
    prompts = _assemble(token_prefix, token_suffix, ctx_g, mix, ctx_s)
    return prompts, tokenized_prompts, aux.reshape(())
